# baseline (speedup 1.0000x reference)
import jax
import jax.numpy as jnp
from jax import lax
from jax.experimental import pallas as pl
from jax.experimental.pallas import tpu as pltpu
from jax.experimental.pallas import tpu_sc as plsc

H = 32
W = 32
_LANES = 16
_SC_WORKERS = 32


def _full_sc_kernel(flat_hbm, row_hbm, col_hbm, r_hbm, c_hbm, out_hbm,
                    idx_r, idx_c, pos_v, rows_c, io, sem_r, sem_c):
    chunk = pos_v.shape[0]
    d = pos_v.shape[1]
    nb = flat_hbm.shape[0] // (chunk * _SC_WORKERS)
    seq = chunk * _SC_WORKERS
    wid = lax.axis_index("s") * 2 + lax.axis_index("c")
    sbase = wid * chunk

    pltpu.sync_copy(r_hbm.at[pl.ds(sbase, chunk)], idx_r)
    pltpu.sync_copy(c_hbm.at[pl.ds(sbase, chunk)], idx_c)
    cp_r = pltpu.async_copy(row_hbm.at[idx_r], pos_v, sem_r)
    cp_c = pltpu.async_copy(col_hbm.at[idx_c], rows_c, sem_c)
    cp_r.wait()
    cp_c.wait()

    def sum_tables(i, carry):
        for j in range(d // _LANES):
            sl = pl.ds(j * _LANES, _LANES)
            pos_v[i, sl] = pos_v[i, sl] + rows_c[i, sl]
        return carry

    lax.fori_loop(0, chunk, sum_tables, 0)

    def batch_body(b, carry):
        rb = b * seq + sbase
        pltpu.sync_copy(flat_hbm.at[pl.ds(rb, chunk)], io)

        def add_pos(i, carry2):
            for j in range(d // _LANES):
                sl = pl.ds(j * _LANES, _LANES)
                io[i, sl] = io[i, sl] + pos_v[i, sl]
            return carry2

        lax.fori_loop(0, chunk, add_pos, 0)
        pltpu.sync_copy(io, out_hbm.at[pl.ds(rb, chunk)])
        return carry

    lax.fori_loop(0, flat_hbm.shape[0] // seq, batch_body, 0)
    del nb


def kernel(output, row_table, col_table, r, c):
    B, S, D = output.shape
    chunk = S // _SC_WORKERS
    flat = output.reshape(B * S, D)
    res = pl.kernel(
        _full_sc_kernel,
        out_type=jax.ShapeDtypeStruct((B * S, D), jnp.float32),
        mesh=plsc.VectorSubcoreMesh(core_axis_name="c", subcore_axis_name="s"),
        scratch_types=[
            pltpu.VMEM((chunk,), jnp.int32),
            pltpu.VMEM((chunk,), jnp.int32),
            pltpu.VMEM((chunk, D), jnp.float32),
            pltpu.VMEM((chunk, D), jnp.float32),
            pltpu.VMEM((chunk, D), jnp.float32),
            pltpu.SemaphoreType.DMA,
            pltpu.SemaphoreType.DMA,
        ],
    )(flat, row_table, col_table, r, c)
    return res.reshape(B, S, D)


# speedup vs baseline: 1.8473x; 1.8473x over previous
import jax
import jax.numpy as jnp
from jax import lax
from jax.experimental import pallas as pl
from jax.experimental.pallas import tpu as pltpu
from jax.experimental.pallas import tpu_sc as plsc

_LANES = 16
_SC_WORKERS = 32


def _full_sc_pipe_kernel(flat_hbm, row_hbm, col_hbm, r_hbm, c_hbm, out_hbm,
                         idx_r, idx_c, pos_v, rows_c,
                         in0, in1, ot0, ot1,
                         sem_r, sem_c, si0, si1, so0, so1):
    chunk = pos_v.shape[0]
    d = pos_v.shape[1]
    seq = chunk * _SC_WORKERS
    nb = flat_hbm.shape[0] // seq
    wid = lax.axis_index("s") * 2 + lax.axis_index("c")
    sbase = wid * chunk

    pltpu.sync_copy(r_hbm.at[pl.ds(sbase, chunk)], idx_r)
    pltpu.sync_copy(c_hbm.at[pl.ds(sbase, chunk)], idx_c)
    cp_r = pltpu.async_copy(row_hbm.at[idx_r], pos_v, sem_r)
    cp_c = pltpu.async_copy(col_hbm.at[idx_c], rows_c, sem_c)
    cp_r.wait()
    cp_c.wait()

    def sum_tables(i, carry):
        for j in range(d // _LANES):
            sl = pl.ds(j * _LANES, _LANES)
            pos_v[i, sl] = pos_v[i, sl] + rows_c[i, sl]
        return carry

    lax.fori_loop(0, chunk, sum_tables, 0)

    ins = (in0, in1)
    ots = (ot0, ot1)
    sis = (si0, si1)
    sos = (so0, so1)

    pltpu.async_copy(flat_hbm.at[pl.ds(sbase, chunk)], in0, si0)
    pltpu.async_copy(flat_hbm.at[pl.ds(seq + sbase, chunk)], in1, si1)

    def outer(g, carry):
        for k in range(2):
            b = 2 * g + k
            rb = b * seq + sbase
            ib, ob, si, so = ins[k], ots[k], sis[k], sos[k]
            pltpu.make_async_copy(flat_hbm.at[pl.ds(rb, chunk)], ib, si).wait()
            @pl.when(b >= 2)
            def _():
                pltpu.make_async_copy(
                    ob, out_hbm.at[pl.ds(rb, chunk)], so).wait()

            def add_pos(i, carry2):
                for j in range(d // _LANES):
                    sl = pl.ds(j * _LANES, _LANES)
                    ob[i, sl] = ib[i, sl] + pos_v[i, sl]
                return carry2

            lax.fori_loop(0, chunk, add_pos, 0)
            pltpu.async_copy(ob, out_hbm.at[pl.ds(rb, chunk)], so)

            @pl.when(b + 2 < nb)
            def _():
                rb2 = (b + 2) * seq + sbase
                pltpu.async_copy(flat_hbm.at[pl.ds(rb2, chunk)], ib, si)
        return carry

    lax.fori_loop(0, nb // 2, outer, 0)
    pltpu.make_async_copy(ot0, out_hbm.at[pl.ds(sbase, chunk)], so0).wait()
    pltpu.make_async_copy(ot1, out_hbm.at[pl.ds(sbase, chunk)], so1).wait()


def sc_pipe(output, row_table, col_table, r, c):
    B, S, D = output.shape
    chunk = S // _SC_WORKERS
    flat = output.reshape(B * S, D)
    res = pl.kernel(
        _full_sc_pipe_kernel,
        out_type=jax.ShapeDtypeStruct((B * S, D), jnp.float32),
        mesh=plsc.VectorSubcoreMesh(core_axis_name="c", subcore_axis_name="s"),
        scratch_types=[
            pltpu.VMEM((chunk,), jnp.int32),
            pltpu.VMEM((chunk,), jnp.int32),
            pltpu.VMEM((chunk, D), jnp.float32),
            pltpu.VMEM((chunk, D), jnp.float32),
            pltpu.VMEM((chunk, D), jnp.float32),
            pltpu.VMEM((chunk, D), jnp.float32),
            pltpu.VMEM((chunk, D), jnp.float32),
            pltpu.VMEM((chunk, D), jnp.float32),
            pltpu.SemaphoreType.DMA,
            pltpu.SemaphoreType.DMA,
            pltpu.SemaphoreType.DMA,
            pltpu.SemaphoreType.DMA,
            pltpu.SemaphoreType.DMA,
            pltpu.SemaphoreType.DMA,
        ],
    )(flat, row_table, col_table, r, c)
    return res.reshape(B, S, D)


def kernel(output, row_table, col_table, r, c):
    return sc_pipe(output, row_table, col_table, r, c)


# speedup vs baseline: 3.0427x; 1.6472x over previous
import jax
import jax.numpy as jnp
from jax.experimental import pallas as pl
from jax.experimental.pallas import tpu as pltpu

H = 32
W = 32


def _add_pos_kernel(r_ref, c_ref, row_tab_ref, col_tab_ref, out_in_ref,
                    out_ref, pos_ref):
    b = pl.program_id(0)

    @pl.when(b == 0)
    def _():
        s = r_ref.shape[0]
        row_oh = (jax.lax.broadcasted_iota(jnp.int32, (s, H), 1)
                  == r_ref[...]).astype(jnp.float32)
        col_oh = (jax.lax.broadcasted_iota(jnp.int32, (s, W), 1)
                  == c_ref[...]).astype(jnp.float32)
        pos_ref[...] = (
            jax.lax.dot(row_oh, row_tab_ref[...],
                        preferred_element_type=jnp.float32)
            + jax.lax.dot(col_oh, col_tab_ref[...],
                          preferred_element_type=jnp.float32))

    s = pos_ref.shape[0]
    nrep = out_ref.shape[0] // s
    for i in range(nrep):
        out_ref[i * s:(i + 1) * s, :] = (
            out_in_ref[i * s:(i + 1) * s, :] + pos_ref[...])


_BATCHES_PER_BLOCK = 8


def kernel(output, row_table, col_table, r, c):
    B, S, D = output.shape
    r2 = r.reshape(S, 1)
    c2 = c.reshape(S, 1)
    flat = output.reshape(B * S, D)
    nb = _BATCHES_PER_BLOCK
    rows = nb * S
    res = pl.pallas_call(
        _add_pos_kernel,
        grid=(B // nb,),
        in_specs=[
            pl.BlockSpec((S, 1), lambda b: (0, 0)),
            pl.BlockSpec((S, 1), lambda b: (0, 0)),
            pl.BlockSpec((H, D), lambda b: (0, 0)),
            pl.BlockSpec((W, D), lambda b: (0, 0)),
            pl.BlockSpec((rows, D), lambda b: (b, 0)),
        ],
        out_specs=pl.BlockSpec((rows, D), lambda b: (b, 0)),
        out_shape=jax.ShapeDtypeStruct((B * S, D), jnp.float32),
        scratch_shapes=[pltpu.VMEM((S, D), jnp.float32)],
    )(r2, c2, row_table, col_table, flat)
    return res.reshape(B, S, D)
